# Initial kernel scaffold; baseline (speedup 1.0000x reference)
import jax
import jax.numpy as jnp
from jax.experimental import pallas as pl


def kernel(word_x, word_edge_index, word_edge_attr, word_batch, sentence_x, sentence_edge_index, sentence_edge_attr, sentence_batch, params):
    raise NotImplementedError("write your pallas kernel here")


# speedup vs baseline: 2.4118x; 2.4118x over previous
import functools

import numpy as np
import jax
import jax.numpy as jnp
from jax import lax
from jax.experimental import pallas as pl
from jax.experimental.pallas import tpu as pltpu
from jax.experimental.pallas import tpu_sc as plsc

NSUB = 16
CHUNK = 128
IBLK = 8
HID = 128
ROW_BLK = 2000


def _cdiv(a, b):
    return (a + b - 1) // b


def _sinusoid_np(n, d):
    pos = np.arange(n)[:, None].astype(np.float32)
    i = np.arange(d)[None, :]
    angle = pos / np.power(10000.0, (2 * (i // 2)) / float(d))
    pe = np.where(i % 2 == 0, np.sin(angle), np.cos(angle))
    return jnp.asarray(pe, jnp.float32)


def _pad_edges(src, dst, attr, nch):
    e = src.shape[0]
    pad = NSUB * nch * CHUNK - e
    src = jnp.pad(src, (0, pad))
    dst = jnp.pad(dst, (0, pad))
    attr = jnp.pad(attr, (0, pad))
    return (src.reshape(NSUB, nch, CHUNK), dst.reshape(NSUB, nch, CHUNK),
            attr.reshape(NSUB, nch * CHUNK))


@functools.lru_cache(maxsize=None)
def _sc_layer(n_nodes, nch_w, nch_s):
    rpt = (n_nodes // NSUB) // 8 * 8
    tail = n_nodes - rpt * NSUB
    mesh = plsc.VectorSubcoreMesh(core_axis_name="c", subcore_axis_name="s")
    nch_max = max(nch_w, nch_s)

    def body(hw, wsrc, wdst, wattr, hs, ssrc, sdst, sattr, zeros,
             m_w, m_f, m_b, acc, srcv, dstv, attrv, rb):
        c = lax.axis_index("c")
        s = lax.axis_index("s")
        own = pl.ds(s * rpt, rpt)
        tl = pl.ds(rpt * NSUB, tail)

        def run(h_hbm, src_hbm, dst_hbm, attr_hbm, out_hbm, nch):
            pltpu.sync_copy(zeros.at[own], acc.at[own])
            if tail:
                @pl.when(s == NSUB - 1)
                def _():
                    pltpu.sync_copy(zeros.at[tl], acc.at[tl])
            plsc.subcore_barrier()

            @pl.loop(0, nch // IBLK)
            def _blk(g):
                pltpu.sync_copy(src_hbm.at[s, pl.ds(g * IBLK, IBLK)], srcv)
                pltpu.sync_copy(dst_hbm.at[s, pl.ds(g * IBLK, IBLK)], dstv)
                pltpu.sync_copy(
                    attr_hbm.at[s, pl.ds(g * IBLK * CHUNK, IBLK * CHUNK)],
                    attrv)

                for jj in range(IBLK):
                    pltpu.sync_copy(h_hbm.at[srcv.at[jj]], rb)

                    @pl.loop(0, CHUNK // 16)
                    def _eg(eg):
                        av16 = attrv[pl.ds(jj * CHUNK + eg * 16, 16)]
                        for l in range(16):
                            bc = lax.gather(
                                av16, jnp.full((16, 1), l, jnp.int32),
                                lax.GatherDimensionNumbers(
                                    offset_dims=(), collapsed_slice_dims=(0,),
                                    start_index_map=(0,)),
                                (1,),
                                mode=lax.GatherScatterMode.PROMISE_IN_BOUNDS)
                            for k in range(HID // 16):
                                ix = (eg * 16 + l, pl.ds(k * 16, 16))
                                rb[ix] = rb[ix] * bc

                    pltpu.sync_copy(rb, acc.at[dstv.at[jj]], add=True)

            plsc.subcore_barrier()
            pltpu.sync_copy(acc.at[own], out_hbm.at[own])
            if tail:
                @pl.when(s == NSUB - 1)
                def _():
                    pltpu.sync_copy(acc.at[tl], out_hbm.at[tl])

        @pl.when(c == 0)
        def _():
            run(hw, wsrc, wdst, wattr, m_w, nch_w)
            plsc.subcore_barrier()
            plsc.subcore_barrier()

        @pl.when(c == 1)
        def _():
            run(hs, ssrc, sdst, sattr, m_f, nch_s)
            run(hs, sdst, ssrc, sattr, m_b, nch_s)

    out_t = [jax.ShapeDtypeStruct((n_nodes, HID), jnp.float32)] * 3
    return pl.kernel(
        body,
        out_type=out_t,
        mesh=mesh,
        scratch_types=[
            pltpu.VMEM_SHARED((n_nodes, HID), jnp.float32),
            pltpu.VMEM((IBLK, CHUNK), jnp.int32),
            pltpu.VMEM((IBLK, CHUNK), jnp.int32),
            pltpu.VMEM((IBLK * CHUNK,), jnp.float32),
            pltpu.VMEM((CHUNK, HID), jnp.float32),
        ],
    )


def _dot(a, b):
    return jnp.dot(a, b, preferred_element_type=jnp.float32)


def _silu(x):
    return x * jax.nn.sigmoid(x)


def _proj_body(wx, sx, win, sin_, pe, hw_o, hs_o):
    hw_o[...] = _dot(wx[...], win[...])
    hs_o[...] = _dot(sx[...], sin_[...]) + pe[...]


def _gru_blk(g, h, wx, wh, b):
    gx = _dot(g, wx) + b
    gh = _dot(h, wh)
    r = jax.nn.sigmoid(gx[:, :HID] + gh[:, :HID])
    z = jax.nn.sigmoid(gx[:, HID:2 * HID] + gh[:, HID:2 * HID])
    n = jnp.tanh(gx[:, 2 * HID:] + r * gh[:, 2 * HID:])
    return (1.0 - z) * n + z * h


def _dense_body(mw, hw, mf, mb, hs,
                wg1, wg2, wwx, wwh, wb,
                sg1, sg2, fwx, fwh, fb, bwx, bwh, bb,
                hw_o, hs_o):
    g = _dot(mw[...], wg1[...]) * _silu(_dot(mw[...], wg2[...]))
    hw_o[...] = _gru_blk(g, hw[...], wwx[...], wwh[...], wb[...])
    gf = _dot(mf[...], sg1[...]) * _silu(_dot(mf[...], sg2[...]))
    gb = _dot(mb[...], sg1[...]) * _silu(_dot(mb[...], sg2[...]))
    hf = _gru_blk(gf, hs[...], fwx[...], fwh[...], fb[...])
    hb = _gru_blk(gb, hs[...], bwx[...], bwh[...], bb[...])
    hs_o[...] = 0.5 * (hf + hb)


def _pool_head_body(hw, hs, wbat, sbat, wout_w, sout_w,
                    fw1, fw2, fb, lng, lnb, c1w, c1b, c2w, c2b,
                    out, wsum, ssum, wcnt, scnt):
    i = pl.program_id(0)
    nblk = pl.num_programs(0)

    @pl.when(i == 0)
    def _():
        wsum[...] = jnp.zeros_like(wsum)
        ssum[...] = jnp.zeros_like(ssum)
        wcnt[...] = jnp.zeros_like(wcnt)
        scnt[...] = jnp.zeros_like(scnt)

    gid = lax.broadcasted_iota(jnp.int32, (64, ROW_BLK), 0)
    yw = _dot(hw[...], wout_w[...])
    ohw = (gid == wbat[0, 0, :][None, :]).astype(jnp.float32)
    wsum[...] += _dot(ohw, yw)
    wcnt[...] += jnp.broadcast_to(jnp.sum(ohw, axis=1, keepdims=True), wcnt.shape)
    ys = _dot(hs[...], sout_w[...])
    ohs = (gid == sbat[0, 0, :][None, :]).astype(jnp.float32)
    ssum[...] += _dot(ohs, ys)
    scnt[...] += jnp.broadcast_to(jnp.sum(ohs, axis=1, keepdims=True), scnt.shape)

    @pl.when(i == nblk - 1)
    def _():
        w = wsum[...] / jnp.maximum(wcnt[...], 1.0)
        so = ssum[...] / jnp.maximum(scnt[...], 1.0)
        alpha = jax.nn.sigmoid(_dot(w, fw1[...]) + _dot(so, fw2[...]) + fb[...])
        fused = alpha * w + (1.0 - alpha) * so
        mu = jnp.mean(fused, axis=-1, keepdims=True)
        xc = fused - mu
        var = jnp.mean(xc * xc, axis=-1, keepdims=True)
        xn = xc * jax.lax.rsqrt(var + 1e-5) * lng[...] + lnb[...]
        xr = jnp.maximum(_dot(xn, c1w[...]) + c1b[...], 0.0)
        out[...] = _dot(xr, c2w[...]) + c2b[...]


def kernel(word_x, word_edge_index, word_edge_attr, word_batch,
           sentence_x, sentence_edge_index, sentence_edge_attr, sentence_batch,
           params):
    p = params
    nw = word_x.shape[0]
    ns = sentence_x.shape[0]
    assert nw == ns and nw % NSUB == 0
    ew = word_edge_index.shape[1]
    es = sentence_edge_index.shape[1]
    ncls = p['c2_w'].shape[1]

    nch_w = _cdiv(ew, NSUB * CHUNK * IBLK) * IBLK
    nch_s = _cdiv(es, NSUB * CHUNK * IBLK) * IBLK
    wsrc, wdst, wattr = _pad_edges(word_edge_index[0], word_edge_index[1],
                                   word_edge_attr, nch_w)
    ssrc, sdst, sattr = _pad_edges(sentence_edge_index[0],
                                   sentence_edge_index[1],
                                   sentence_edge_attr, nch_s)
    zeros = jnp.zeros((nw, HID), jnp.float32)
    pe = _sinusoid_np(ns, HID)

    nblk = nw // ROW_BLK
    grid_rows = lambda: pl.BlockSpec((ROW_BLK, HID), lambda i: (i, 0))
    full = lambda shp: pl.BlockSpec(shp, lambda i: tuple(0 for _ in shp))

    hw, hs = pl.pallas_call(
        _proj_body,
        grid=(nblk,),
        in_specs=[grid_rows(), grid_rows(), full((HID, HID)), full((HID, HID)),
                  grid_rows()],
        out_specs=[grid_rows(), grid_rows()],
        out_shape=[jax.ShapeDtypeStruct((nw, HID), jnp.float32)] * 2,
    )(word_x, sentence_x, p['w_in'], p['s_in'], pe)

    sc = _sc_layer(nw, nch_w, nch_s)
    dense = pl.pallas_call(
        _dense_body,
        grid=(nblk,),
        in_specs=[grid_rows()] * 5 + [
            full((HID, HID)), full((HID, HID)),
            full((HID, 3 * HID)), full((HID, 3 * HID)), full((1, 3 * HID)),
            full((HID, HID)), full((HID, HID)),
            full((HID, 3 * HID)), full((HID, 3 * HID)), full((1, 3 * HID)),
            full((HID, 3 * HID)), full((HID, 3 * HID)), full((1, 3 * HID)),
        ],
        out_specs=[grid_rows(), grid_rows()],
        out_shape=[jax.ShapeDtypeStruct((nw, HID), jnp.float32)] * 2,
    )

    wb = p['w_gru_b'].reshape(1, 3 * HID)
    fbias = p['s_gru_b_f'].reshape(1, 3 * HID)
    bbias = p['s_gru_b_b'].reshape(1, 3 * HID)
    for _ in range(3):
        m_w, m_f, m_b = sc(hw, wsrc, wdst, wattr, hs, ssrc, sdst, sattr, zeros)
        hw, hs = dense(m_w, hw, m_f, m_b, hs,
                       p['w_g1'], p['w_g2'], p['w_gru_wx'], p['w_gru_wh'], wb,
                       p['s_g1'], p['s_g2'],
                       p['s_gru_wx_f'], p['s_gru_wh_f'], fbias,
                       p['s_gru_wx_b'], p['s_gru_wh_b'], bbias)

    c2w = jnp.zeros((HID, HID), jnp.float32).at[:, :ncls].set(p['c2_w'])
    c2b = jnp.zeros((1, HID), jnp.float32).at[0, :ncls].set(p['c2_b'])
    wbat = word_batch.reshape(nblk, 1, ROW_BLK)
    sbat = sentence_batch.reshape(nblk, 1, ROW_BLK)
    bat_spec = pl.BlockSpec((1, 1, ROW_BLK), lambda i: (i, 0, 0))

    out = pl.pallas_call(
        _pool_head_body,
        grid=(nblk,),
        in_specs=[grid_rows(), grid_rows(), bat_spec, bat_spec,
                  full((HID, HID)), full((HID, HID)),
                  full((HID, HID)), full((HID, HID)), full((1, HID)),
                  full((1, HID)), full((1, HID)),
                  full((HID, HID)), full((1, HID)),
                  full((HID, HID)), full((1, HID))],
        out_specs=pl.BlockSpec((64, HID), lambda i: (0, 0)),
        out_shape=jax.ShapeDtypeStruct((64, HID), jnp.float32),
        scratch_shapes=[pltpu.VMEM((64, HID), jnp.float32)] * 4,
    )(hw, hs, wbat, sbat, p['w_out'], p['s_out'],
      p['fuse_w'][:HID], p['fuse_w'][HID:],
      p['fuse_b'].reshape(1, HID),
      p['ln_g'].reshape(1, HID), p['ln_b'].reshape(1, HID),
      p['c1_w'], p['c1_b'].reshape(1, HID), c2w, c2b)

    return out[:, :ncls]


# speedup vs baseline: 2.9670x; 1.2302x over previous
import functools

import numpy as np
import jax
import jax.numpy as jnp
from jax import lax
from jax.experimental import pallas as pl
from jax.experimental.pallas import tpu as pltpu
from jax.experimental.pallas import tpu_sc as plsc

NSUB = 16
CHUNK = 64
IBLK = 8
NBUF = 4
PF = 2
HID = 128
ROW_BLK = 2000


def _cdiv(a, b):
    return (a + b - 1) // b


def _sinusoid_np(n, d):
    pos = np.arange(n)[:, None].astype(np.float32)
    i = np.arange(d)[None, :]
    angle = pos / np.power(10000.0, (2 * (i // 2)) / float(d))
    pe = np.where(i % 2 == 0, np.sin(angle), np.cos(angle))
    return jnp.asarray(pe, jnp.float32)


def _pad_edges(src, dst, attr, nch):
    e = src.shape[0]
    pad = NSUB * nch * CHUNK - e
    src = jnp.pad(src, (0, pad))
    dst = jnp.pad(dst, (0, pad))
    attr = jnp.pad(attr, (0, pad))
    return (src.reshape(NSUB, nch, CHUNK), dst.reshape(NSUB, nch, CHUNK),
            attr.reshape(NSUB, nch * CHUNK))


_GDN = lax.GatherDimensionNumbers(
    offset_dims=(), collapsed_slice_dims=(0,), start_index_map=(0,))


@functools.lru_cache(maxsize=None)
def _sc_layer(n_nodes, nch_w, nch_s):
    rpt = (n_nodes // NSUB) // 8 * 8
    tail = n_nodes - rpt * NSUB
    mesh = plsc.VectorSubcoreMesh(core_axis_name="c", subcore_axis_name="s")
    nch_max = max(nch_w, nch_s)

    def body(hw, wsrc, wdst, wattr, hs, ssrc, sdst, sattr, zeros,
             m_w, m_f, m_b, acc,
             srcv0, srcv1, dstv0, dstv1, attrv0, attrv1,
             rb0, rb1, rb2, rb3,
             gsem0, gsem1, gsem2, gsem3,
             ssem0, ssem1, ssem2, ssem3,
             isem0, isem1):
        c = lax.axis_index("c")
        s = lax.axis_index("s")
        own = pl.ds(s * rpt, rpt)
        tl = pl.ds(rpt * NSUB, tail)
        srcs, dsts, attrs = [srcv0, srcv1], [dstv0, dstv1], [attrv0, attrv1]
        rbs = [rb0, rb1, rb2, rb3]
        gsems = [gsem0, gsem1, gsem2, gsem3]
        ssems = [ssem0, ssem1, ssem2, ssem3]
        isems = [isem0, isem1]

        def run(h_hbm, src_hbm, dst_hbm, attr_hbm, out_hbm, nch):
            nblk = nch // IBLK

            def stage_copies(g1, sbn):
                return [
                    pltpu.make_async_copy(
                        src_hbm.at[s, pl.ds(g1 * IBLK, IBLK)],
                        srcs[sbn], isems[sbn]),
                    pltpu.make_async_copy(
                        dst_hbm.at[s, pl.ds(g1 * IBLK, IBLK)],
                        dsts[sbn], isems[sbn]),
                    pltpu.make_async_copy(
                        attr_hbm.at[s, pl.ds(g1 * IBLK * CHUNK, IBLK * CHUNK)],
                        attrs[sbn], isems[sbn]),
                ]

            pltpu.sync_copy(zeros.at[own], acc.at[own])
            if tail:
                @pl.when(s == NSUB - 1)
                def _():
                    pltpu.sync_copy(zeros.at[tl], acc.at[tl])
            plsc.subcore_barrier()

            pltpu.sync_copy(src_hbm.at[s, pl.ds(0, IBLK)], srcs[0])
            pltpu.sync_copy(dst_hbm.at[s, pl.ds(0, IBLK)], dsts[0])
            pltpu.sync_copy(attr_hbm.at[s, pl.ds(0, IBLK * CHUNK)], attrs[0])
            for jj in range(PF):
                pltpu.async_copy(h_hbm.at[srcs[0].at[jj]], rbs[jj], gsems[jj])

            def process_block(g, sbi):
                sb, sbn = sbi, 1 - sbi
                have_next = g + 1 < nblk

                @pl.when(have_next)
                def _():
                    for d in stage_copies(g + 1, sbn):
                        d.start()

                for jj in range(IBLK):
                    j = g * IBLK + jj
                    b = jj % NBUF
                    pltpu.make_async_copy(
                        h_hbm.at[srcs[sb].at[jj]], rbs[b], gsems[b]).wait()

                    @pl.loop(0, CHUNK // 16)
                    def _eg(eg):
                        av16 = attrs[sb][pl.ds(jj * CHUNK + eg * 16, 16)]

                        @pl.loop(0, 16, unroll=4)
                        def _l(l):
                            bc = lax.gather(
                                av16, jnp.full((16, 1), l, jnp.int32),
                                _GDN, (1,),
                                mode=lax.GatherScatterMode.PROMISE_IN_BOUNDS)
                            for k in range(HID // 16):
                                ix = (eg * 16 + l, pl.ds(k * 16, 16))
                                rbs[b][ix] = rbs[b][ix] * bc

                    pltpu.async_copy(rbs[b], acc.at[dsts[sb].at[jj]],
                                     ssems[b], add=True)

                    if jj == IBLK - PF:
                        @pl.when(have_next)
                        def _():
                            for d in stage_copies(g + 1, sbn):
                                d.wait()

                    jn = j + PF
                    jjn = jj + PF
                    bn = jjn % NBUF
                    nsrc = (srcs[sb].at[jjn] if jjn < IBLK
                            else srcs[sbn].at[jjn - IBLK])

                    @pl.when((jn < nch) & (j >= NBUF - PF))
                    def _():
                        pltpu.make_async_copy(
                            rbs[bn], acc.at[dsts[sb].at[jj]],
                            ssems[bn]).wait()

                    @pl.when(jn < nch)
                    def _():
                        pltpu.async_copy(h_hbm.at[nsrc], rbs[bn], gsems[bn])

            @pl.loop(0, nblk // 2)
            def _g2(g2):
                process_block(g2 * 2, 0)
                process_block(g2 * 2 + 1, 1)

            for b in range(NBUF):
                pltpu.make_async_copy(
                    rbs[b], acc.at[dsts[0].at[0]], ssems[b]).wait()

            plsc.subcore_barrier()
            pltpu.sync_copy(acc.at[own], out_hbm.at[own])
            if tail:
                @pl.when(s == NSUB - 1)
                def _():
                    pltpu.sync_copy(acc.at[tl], out_hbm.at[tl])

        @pl.when(c == 0)
        def _():
            run(hw, wsrc, wdst, wattr, m_w, nch_w)
            plsc.subcore_barrier()
            plsc.subcore_barrier()

        @pl.when(c == 1)
        def _():
            run(hs, ssrc, sdst, sattr, m_f, nch_s)
            run(hs, sdst, ssrc, sattr, m_b, nch_s)

    out_t = [jax.ShapeDtypeStruct((n_nodes, HID), jnp.float32)] * 3
    return pl.kernel(
        body,
        out_type=out_t,
        mesh=mesh,
        scratch_types=(
            [pltpu.VMEM_SHARED((n_nodes, HID), jnp.float32)]
            + [pltpu.VMEM((IBLK, CHUNK), jnp.int32)] * 4
            + [pltpu.VMEM((IBLK * CHUNK,), jnp.float32)] * 2
            + [pltpu.VMEM((CHUNK, HID), jnp.float32)] * NBUF
            + [pltpu.SemaphoreType.DMA] * (2 * NBUF + 2)
        ),
    )


def _dot(a, b):
    return jnp.dot(a, b, preferred_element_type=jnp.float32)


def _silu(x):
    return x * jax.nn.sigmoid(x)


def _proj_body(wx, sx, win, sin_, pe, hw_o, hs_o):
    hw_o[...] = _dot(wx[...], win[...])
    hs_o[...] = _dot(sx[...], sin_[...]) + pe[...]


def _gru_blk(g, h, wx, wh, b):
    gx = _dot(g, wx) + b
    gh = _dot(h, wh)
    r = jax.nn.sigmoid(gx[:, :HID] + gh[:, :HID])
    z = jax.nn.sigmoid(gx[:, HID:2 * HID] + gh[:, HID:2 * HID])
    n = jnp.tanh(gx[:, 2 * HID:] + r * gh[:, 2 * HID:])
    return (1.0 - z) * n + z * h


def _dense_body(mw, hw, mf, mb, hs,
                wg1, wg2, wwx, wwh, wb,
                sg1, sg2, fwx, fwh, fb, bwx, bwh, bb,
                hw_o, hs_o):
    g = _dot(mw[...], wg1[...]) * _silu(_dot(mw[...], wg2[...]))
    hw_o[...] = _gru_blk(g, hw[...], wwx[...], wwh[...], wb[...])
    gf = _dot(mf[...], sg1[...]) * _silu(_dot(mf[...], sg2[...]))
    gb = _dot(mb[...], sg1[...]) * _silu(_dot(mb[...], sg2[...]))
    hf = _gru_blk(gf, hs[...], fwx[...], fwh[...], fb[...])
    hb = _gru_blk(gb, hs[...], bwx[...], bwh[...], bb[...])
    hs_o[...] = 0.5 * (hf + hb)


def _pool_head_body(hw, hs, wbat, sbat, wout_w, sout_w,
                    fw1, fw2, fb, lng, lnb, c1w, c1b, c2w, c2b,
                    out, wsum, ssum, wcnt, scnt):
    i = pl.program_id(0)
    nblk = pl.num_programs(0)

    @pl.when(i == 0)
    def _():
        wsum[...] = jnp.zeros_like(wsum)
        ssum[...] = jnp.zeros_like(ssum)
        wcnt[...] = jnp.zeros_like(wcnt)
        scnt[...] = jnp.zeros_like(scnt)

    gid = lax.broadcasted_iota(jnp.int32, (64, ROW_BLK), 0)
    yw = _dot(hw[...], wout_w[...])
    ohw = (gid == wbat[0, 0, :][None, :]).astype(jnp.float32)
    wsum[...] += _dot(ohw, yw)
    wcnt[...] += jnp.broadcast_to(jnp.sum(ohw, axis=1, keepdims=True), wcnt.shape)
    ys = _dot(hs[...], sout_w[...])
    ohs = (gid == sbat[0, 0, :][None, :]).astype(jnp.float32)
    ssum[...] += _dot(ohs, ys)
    scnt[...] += jnp.broadcast_to(jnp.sum(ohs, axis=1, keepdims=True), scnt.shape)

    @pl.when(i == nblk - 1)
    def _():
        w = wsum[...] / jnp.maximum(wcnt[...], 1.0)
        so = ssum[...] / jnp.maximum(scnt[...], 1.0)
        alpha = jax.nn.sigmoid(_dot(w, fw1[...]) + _dot(so, fw2[...]) + fb[...])
        fused = alpha * w + (1.0 - alpha) * so
        mu = jnp.mean(fused, axis=-1, keepdims=True)
        xc = fused - mu
        var = jnp.mean(xc * xc, axis=-1, keepdims=True)
        xn = xc * jax.lax.rsqrt(var + 1e-5) * lng[...] + lnb[...]
        xr = jnp.maximum(_dot(xn, c1w[...]) + c1b[...], 0.0)
        out[...] = _dot(xr, c2w[...]) + c2b[...]


def kernel(word_x, word_edge_index, word_edge_attr, word_batch,
           sentence_x, sentence_edge_index, sentence_edge_attr, sentence_batch,
           params):
    p = params
    nw = word_x.shape[0]
    ns = sentence_x.shape[0]
    assert nw == ns and nw % NSUB == 0
    ew = word_edge_index.shape[1]
    es = sentence_edge_index.shape[1]
    ncls = p['c2_w'].shape[1]

    nch_w = _cdiv(ew, NSUB * CHUNK * 2 * IBLK) * 2 * IBLK
    nch_s = _cdiv(es, NSUB * CHUNK * 2 * IBLK) * 2 * IBLK
    wsrc, wdst, wattr = _pad_edges(word_edge_index[0], word_edge_index[1],
                                   word_edge_attr, nch_w)
    ssrc, sdst, sattr = _pad_edges(sentence_edge_index[0],
                                   sentence_edge_index[1],
                                   sentence_edge_attr, nch_s)
    zeros = jnp.zeros((nw, HID), jnp.float32)
    pe = _sinusoid_np(ns, HID)

    nblk = nw // ROW_BLK
    grid_rows = lambda: pl.BlockSpec((ROW_BLK, HID), lambda i: (i, 0))
    full = lambda shp: pl.BlockSpec(shp, lambda i: tuple(0 for _ in shp))

    hw, hs = pl.pallas_call(
        _proj_body,
        grid=(nblk,),
        in_specs=[grid_rows(), grid_rows(), full((HID, HID)), full((HID, HID)),
                  grid_rows()],
        out_specs=[grid_rows(), grid_rows()],
        out_shape=[jax.ShapeDtypeStruct((nw, HID), jnp.float32)] * 2,
    )(word_x, sentence_x, p['w_in'], p['s_in'], pe)

    sc = _sc_layer(nw, nch_w, nch_s)
    dense = pl.pallas_call(
        _dense_body,
        grid=(nblk,),
        in_specs=[grid_rows()] * 5 + [
            full((HID, HID)), full((HID, HID)),
            full((HID, 3 * HID)), full((HID, 3 * HID)), full((1, 3 * HID)),
            full((HID, HID)), full((HID, HID)),
            full((HID, 3 * HID)), full((HID, 3 * HID)), full((1, 3 * HID)),
            full((HID, 3 * HID)), full((HID, 3 * HID)), full((1, 3 * HID)),
        ],
        out_specs=[grid_rows(), grid_rows()],
        out_shape=[jax.ShapeDtypeStruct((nw, HID), jnp.float32)] * 2,
    )

    wb = p['w_gru_b'].reshape(1, 3 * HID)
    fbias = p['s_gru_b_f'].reshape(1, 3 * HID)
    bbias = p['s_gru_b_b'].reshape(1, 3 * HID)
    for _ in range(3):
        m_w, m_f, m_b = sc(hw, wsrc, wdst, wattr, hs, ssrc, sdst, sattr, zeros)
        hw, hs = dense(m_w, hw, m_f, m_b, hs,
                       p['w_g1'], p['w_g2'], p['w_gru_wx'], p['w_gru_wh'], wb,
                       p['s_g1'], p['s_g2'],
                       p['s_gru_wx_f'], p['s_gru_wh_f'], fbias,
                       p['s_gru_wx_b'], p['s_gru_wh_b'], bbias)

    c2w = jnp.zeros((HID, HID), jnp.float32).at[:, :ncls].set(p['c2_w'])
    c2b = jnp.zeros((1, HID), jnp.float32).at[0, :ncls].set(p['c2_b'])
    wbat = word_batch.reshape(nblk, 1, ROW_BLK)
    sbat = sentence_batch.reshape(nblk, 1, ROW_BLK)
    bat_spec = pl.BlockSpec((1, 1, ROW_BLK), lambda i: (i, 0, 0))

    out = pl.pallas_call(
        _pool_head_body,
        grid=(nblk,),
        in_specs=[grid_rows(), grid_rows(), bat_spec, bat_spec,
                  full((HID, HID)), full((HID, HID)),
                  full((HID, HID)), full((HID, HID)), full((1, HID)),
                  full((1, HID)), full((1, HID)),
                  full((HID, HID)), full((1, HID)),
                  full((HID, HID)), full((1, HID))],
        out_specs=pl.BlockSpec((64, HID), lambda i: (0, 0)),
        out_shape=jax.ShapeDtypeStruct((64, HID), jnp.float32),
        scratch_shapes=[pltpu.VMEM((64, HID), jnp.float32)] * 4,
    )(hw, hs, wbat, sbat, p['w_out'], p['s_out'],
      p['fuse_w'][:HID], p['fuse_w'][HID:],
      p['fuse_b'].reshape(1, HID),
      p['ln_g'].reshape(1, HID), p['ln_b'].reshape(1, HID),
      p['c1_w'], p['c1_b'].reshape(1, HID), c2w, c2b)

    return out[:, :ncls]


# speedup vs baseline: 3.3090x; 1.1153x over previous
import functools

import numpy as np
import jax
import jax.numpy as jnp
from jax import lax
from jax.experimental import pallas as pl
from jax.experimental.pallas import tpu as pltpu
from jax.experimental.pallas import tpu_sc as plsc

NSUB = 16
CHUNK = 128
IBLK = 4
NBUF = 2
PF = 2
HID = 128
ROW_BLK = 2000


def _cdiv(a, b):
    return (a + b - 1) // b


def _sinusoid_np(n, d):
    pos = np.arange(n)[:, None].astype(np.float32)
    i = np.arange(d)[None, :]
    angle = pos / np.power(10000.0, (2 * (i // 2)) / float(d))
    pe = np.where(i % 2 == 0, np.sin(angle), np.cos(angle))
    return jnp.asarray(pe, jnp.float32)


def _pad_edges(src, dst, attr, nch):
    e = src.shape[0]
    pad = NSUB * nch * CHUNK - e
    src = jnp.pad(src, (0, pad))
    dst = jnp.pad(dst, (0, pad))
    attr = jnp.pad(attr, (0, pad))
    return (src.reshape(NSUB, nch, CHUNK), dst.reshape(NSUB, nch, CHUNK),
            attr.reshape(NSUB, nch * CHUNK))


_GDN = lax.GatherDimensionNumbers(
    offset_dims=(), collapsed_slice_dims=(0,), start_index_map=(0,))


@functools.lru_cache(maxsize=None)
def _sc_layer(n_nodes, nch_w, nch_s):
    rpt = (n_nodes // NSUB) // 8 * 8
    tail = n_nodes - rpt * NSUB
    mesh = plsc.VectorSubcoreMesh(core_axis_name="c", subcore_axis_name="s")
    nch_max = max(nch_w, nch_s)

    def body(*refs):
        (hw, wsrc, wdst, wattr, hs, ssrc, sdst, sattr, zeros,
         m_w, m_f, m_b, acc) = refs[:13]
        rest = list(refs[13:])
        srcs = [rest.pop(0), rest.pop(0)]
        dsts = [rest.pop(0), rest.pop(0)]
        attrs = [rest.pop(0), rest.pop(0)]
        rbs = [rest.pop(0) for _ in range(NBUF)]
        gsems = [rest.pop(0) for _ in range(NBUF)]
        ssems = [rest.pop(0) for _ in range(NBUF)]
        isems = [rest.pop(0), rest.pop(0)]
        c = lax.axis_index("c")
        s = lax.axis_index("s")
        own = pl.ds(s * rpt, rpt)
        tl = pl.ds(rpt * NSUB, tail)

        def run(h_hbm, src_hbm, dst_hbm, attr_hbm, out_hbm, nch):
            nblk = nch // IBLK

            def stage_copies(g1, sbn):
                return [
                    pltpu.make_async_copy(
                        src_hbm.at[s, pl.ds(g1 * IBLK, IBLK)],
                        srcs[sbn], isems[sbn]),
                    pltpu.make_async_copy(
                        dst_hbm.at[s, pl.ds(g1 * IBLK, IBLK)],
                        dsts[sbn], isems[sbn]),
                    pltpu.make_async_copy(
                        attr_hbm.at[s, pl.ds(g1 * IBLK * CHUNK, IBLK * CHUNK)],
                        attrs[sbn], isems[sbn]),
                ]

            pltpu.sync_copy(zeros.at[own], acc.at[own])
            if tail:
                @pl.when(s == NSUB - 1)
                def _():
                    pltpu.sync_copy(zeros.at[tl], acc.at[tl])
            plsc.subcore_barrier()

            pltpu.sync_copy(src_hbm.at[s, pl.ds(0, IBLK)], srcs[0])
            pltpu.sync_copy(dst_hbm.at[s, pl.ds(0, IBLK)], dsts[0])
            pltpu.sync_copy(attr_hbm.at[s, pl.ds(0, IBLK * CHUNK)], attrs[0])
            for jj in range(PF):
                pltpu.async_copy(h_hbm.at[srcs[0].at[jj]], rbs[jj], gsems[jj])

            def process_block(g, sbi):
                sb, sbn = sbi, 1 - sbi
                have_next = g + 1 < nblk

                @pl.when(have_next)
                def _():
                    for d in stage_copies(g + 1, sbn):
                        d.start()

                for jj in range(IBLK):
                    j = g * IBLK + jj
                    b = jj % NBUF
                    pltpu.make_async_copy(
                        h_hbm.at[srcs[sb].at[jj]], rbs[b], gsems[b]).wait()

                    @pl.loop(0, CHUNK // 16)
                    def _eg(eg):
                        av16 = attrs[sb][pl.ds(jj * CHUNK + eg * 16, 16)]

                        @pl.loop(0, 16, unroll=4)
                        def _l(l):
                            bc = lax.gather(
                                av16, jnp.full((16, 1), l, jnp.int32),
                                _GDN, (1,),
                                mode=lax.GatherScatterMode.PROMISE_IN_BOUNDS)
                            for k in range(HID // 16):
                                ix = (eg * 16 + l, pl.ds(k * 16, 16))
                                rbs[b][ix] = rbs[b][ix] * bc

                    pltpu.async_copy(rbs[b], acc.at[dsts[sb].at[jj]],
                                     ssems[b], add=True)

                    if jj == IBLK - PF:
                        @pl.when(have_next)
                        def _():
                            for d in stage_copies(g + 1, sbn):
                                d.wait()

                    jn = j + PF
                    jjn = jj + PF
                    bn = jjn % NBUF
                    nsrc = (srcs[sb].at[jjn] if jjn < IBLK
                            else srcs[sbn].at[jjn - IBLK])

                    @pl.when((jn < nch) & (j >= NBUF - PF))
                    def _():
                        pltpu.make_async_copy(
                            rbs[bn], acc.at[dsts[sb].at[jj]],
                            ssems[bn]).wait()

                    @pl.when(jn < nch)
                    def _():
                        pltpu.async_copy(h_hbm.at[nsrc], rbs[bn], gsems[bn])

            @pl.loop(0, nblk // 2)
            def _g2(g2):
                process_block(g2 * 2, 0)
                process_block(g2 * 2 + 1, 1)

            for b in range(NBUF):
                pltpu.make_async_copy(
                    rbs[b], acc.at[dsts[0].at[0]], ssems[b]).wait()

            plsc.subcore_barrier()
            pltpu.sync_copy(acc.at[own], out_hbm.at[own])
            if tail:
                @pl.when(s == NSUB - 1)
                def _():
                    pltpu.sync_copy(acc.at[tl], out_hbm.at[tl])

        @pl.when(c == 0)
        def _():
            run(hw, wsrc, wdst, wattr, m_w, nch_w)
            plsc.subcore_barrier()
            plsc.subcore_barrier()

        @pl.when(c == 1)
        def _():
            run(hs, ssrc, sdst, sattr, m_f, nch_s)
            run(hs, sdst, ssrc, sattr, m_b, nch_s)

    out_t = [jax.ShapeDtypeStruct((n_nodes, HID), jnp.float32)] * 3
    return pl.kernel(
        body,
        out_type=out_t,
        mesh=mesh,
        scratch_types=(
            [pltpu.VMEM_SHARED((n_nodes, HID), jnp.float32)]
            + [pltpu.VMEM((IBLK, CHUNK), jnp.int32)] * 4
            + [pltpu.VMEM((IBLK * CHUNK,), jnp.float32)] * 2
            + [pltpu.VMEM((CHUNK, HID), jnp.float32)] * NBUF
            + [pltpu.SemaphoreType.DMA] * (2 * NBUF + 2)
        ),
    )


def _dot(a, b):
    return jnp.dot(a, b, preferred_element_type=jnp.float32)


def _silu(x):
    return x * jax.nn.sigmoid(x)


def _proj_body(wx, sx, win, sin_, pe, hw_o, hs_o):
    hw_o[...] = _dot(wx[...], win[...])
    hs_o[...] = _dot(sx[...], sin_[...]) + pe[...]


def _gru_blk(g, h, wx, wh, b):
    gx = _dot(g, wx) + b
    gh = _dot(h, wh)
    r = jax.nn.sigmoid(gx[:, :HID] + gh[:, :HID])
    z = jax.nn.sigmoid(gx[:, HID:2 * HID] + gh[:, HID:2 * HID])
    n = jnp.tanh(gx[:, 2 * HID:] + r * gh[:, 2 * HID:])
    return (1.0 - z) * n + z * h


def _dense_body(mw, hw, mf, mb, hs,
                wg1, wg2, wwx, wwh, wb,
                sg1, sg2, fwx, fwh, fb, bwx, bwh, bb,
                hw_o, hs_o):
    g = _dot(mw[...], wg1[...]) * _silu(_dot(mw[...], wg2[...]))
    hw_o[...] = _gru_blk(g, hw[...], wwx[...], wwh[...], wb[...])
    gf = _dot(mf[...], sg1[...]) * _silu(_dot(mf[...], sg2[...]))
    gb = _dot(mb[...], sg1[...]) * _silu(_dot(mb[...], sg2[...]))
    hf = _gru_blk(gf, hs[...], fwx[...], fwh[...], fb[...])
    hb = _gru_blk(gb, hs[...], bwx[...], bwh[...], bb[...])
    hs_o[...] = 0.5 * (hf + hb)


def _pool_head_body(hw, hs, wbat, sbat, wout_w, sout_w,
                    fw1, fw2, fb, lng, lnb, c1w, c1b, c2w, c2b,
                    out, wsum, ssum, wcnt, scnt):
    i = pl.program_id(0)
    nblk = pl.num_programs(0)

    @pl.when(i == 0)
    def _():
        wsum[...] = jnp.zeros_like(wsum)
        ssum[...] = jnp.zeros_like(ssum)
        wcnt[...] = jnp.zeros_like(wcnt)
        scnt[...] = jnp.zeros_like(scnt)

    gid = lax.broadcasted_iota(jnp.int32, (64, ROW_BLK), 0)
    yw = _dot(hw[...], wout_w[...])
    ohw = (gid == wbat[0, 0, :][None, :]).astype(jnp.float32)
    wsum[...] += _dot(ohw, yw)
    wcnt[...] += jnp.broadcast_to(jnp.sum(ohw, axis=1, keepdims=True), wcnt.shape)
    ys = _dot(hs[...], sout_w[...])
    ohs = (gid == sbat[0, 0, :][None, :]).astype(jnp.float32)
    ssum[...] += _dot(ohs, ys)
    scnt[...] += jnp.broadcast_to(jnp.sum(ohs, axis=1, keepdims=True), scnt.shape)

    @pl.when(i == nblk - 1)
    def _():
        w = wsum[...] / jnp.maximum(wcnt[...], 1.0)
        so = ssum[...] / jnp.maximum(scnt[...], 1.0)
        alpha = jax.nn.sigmoid(_dot(w, fw1[...]) + _dot(so, fw2[...]) + fb[...])
        fused = alpha * w + (1.0 - alpha) * so
        mu = jnp.mean(fused, axis=-1, keepdims=True)
        xc = fused - mu
        var = jnp.mean(xc * xc, axis=-1, keepdims=True)
        xn = xc * jax.lax.rsqrt(var + 1e-5) * lng[...] + lnb[...]
        xr = jnp.maximum(_dot(xn, c1w[...]) + c1b[...], 0.0)
        out[...] = _dot(xr, c2w[...]) + c2b[...]


def kernel(word_x, word_edge_index, word_edge_attr, word_batch,
           sentence_x, sentence_edge_index, sentence_edge_attr, sentence_batch,
           params):
    p = params
    nw = word_x.shape[0]
    ns = sentence_x.shape[0]
    assert nw == ns and nw % NSUB == 0
    ew = word_edge_index.shape[1]
    es = sentence_edge_index.shape[1]
    ncls = p['c2_w'].shape[1]

    nch_w = _cdiv(ew, NSUB * CHUNK * 2 * IBLK) * 2 * IBLK
    nch_s = _cdiv(es, NSUB * CHUNK * 2 * IBLK) * 2 * IBLK
    wsrc, wdst, wattr = _pad_edges(word_edge_index[0], word_edge_index[1],
                                   word_edge_attr, nch_w)
    ssrc, sdst, sattr = _pad_edges(sentence_edge_index[0],
                                   sentence_edge_index[1],
                                   sentence_edge_attr, nch_s)
    zeros = jnp.zeros((nw, HID), jnp.float32)
    pe = _sinusoid_np(ns, HID)

    nblk = nw // ROW_BLK
    grid_rows = lambda: pl.BlockSpec((ROW_BLK, HID), lambda i: (i, 0))
    full = lambda shp: pl.BlockSpec(shp, lambda i: tuple(0 for _ in shp))

    hw, hs = pl.pallas_call(
        _proj_body,
        grid=(nblk,),
        in_specs=[grid_rows(), grid_rows(), full((HID, HID)), full((HID, HID)),
                  grid_rows()],
        out_specs=[grid_rows(), grid_rows()],
        out_shape=[jax.ShapeDtypeStruct((nw, HID), jnp.float32)] * 2,
    )(word_x, sentence_x, p['w_in'], p['s_in'], pe)

    sc = _sc_layer(nw, nch_w, nch_s)
    dense = pl.pallas_call(
        _dense_body,
        grid=(nblk,),
        in_specs=[grid_rows()] * 5 + [
            full((HID, HID)), full((HID, HID)),
            full((HID, 3 * HID)), full((HID, 3 * HID)), full((1, 3 * HID)),
            full((HID, HID)), full((HID, HID)),
            full((HID, 3 * HID)), full((HID, 3 * HID)), full((1, 3 * HID)),
            full((HID, 3 * HID)), full((HID, 3 * HID)), full((1, 3 * HID)),
        ],
        out_specs=[grid_rows(), grid_rows()],
        out_shape=[jax.ShapeDtypeStruct((nw, HID), jnp.float32)] * 2,
    )

    wb = p['w_gru_b'].reshape(1, 3 * HID)
    fbias = p['s_gru_b_f'].reshape(1, 3 * HID)
    bbias = p['s_gru_b_b'].reshape(1, 3 * HID)
    for _ in range(3):
        m_w, m_f, m_b = sc(hw, wsrc, wdst, wattr, hs, ssrc, sdst, sattr, zeros)
        hw, hs = dense(m_w, hw, m_f, m_b, hs,
                       p['w_g1'], p['w_g2'], p['w_gru_wx'], p['w_gru_wh'], wb,
                       p['s_g1'], p['s_g2'],
                       p['s_gru_wx_f'], p['s_gru_wh_f'], fbias,
                       p['s_gru_wx_b'], p['s_gru_wh_b'], bbias)

    c2w = jnp.zeros((HID, HID), jnp.float32).at[:, :ncls].set(p['c2_w'])
    c2b = jnp.zeros((1, HID), jnp.float32).at[0, :ncls].set(p['c2_b'])
    wbat = word_batch.reshape(nblk, 1, ROW_BLK)
    sbat = sentence_batch.reshape(nblk, 1, ROW_BLK)
    bat_spec = pl.BlockSpec((1, 1, ROW_BLK), lambda i: (i, 0, 0))

    out = pl.pallas_call(
        _pool_head_body,
        grid=(nblk,),
        in_specs=[grid_rows(), grid_rows(), bat_spec, bat_spec,
                  full((HID, HID)), full((HID, HID)),
                  full((HID, HID)), full((HID, HID)), full((1, HID)),
                  full((1, HID)), full((1, HID)),
                  full((HID, HID)), full((1, HID)),
                  full((HID, HID)), full((1, HID))],
        out_specs=pl.BlockSpec((64, HID), lambda i: (0, 0)),
        out_shape=jax.ShapeDtypeStruct((64, HID), jnp.float32),
        scratch_shapes=[pltpu.VMEM((64, HID), jnp.float32)] * 4,
    )(hw, hs, wbat, sbat, p['w_out'], p['s_out'],
      p['fuse_w'][:HID], p['fuse_w'][HID:],
      p['fuse_b'].reshape(1, HID),
      p['ln_g'].reshape(1, HID), p['ln_b'].reshape(1, HID),
      p['c1_w'], p['c1_b'].reshape(1, HID), c2w, c2b)

    return out[:, :ncls]


# speedup vs baseline: 4.6848x; 1.4158x over previous
import functools

import numpy as np
import jax
import jax.numpy as jnp
from jax import lax
from jax.experimental import pallas as pl
from jax.experimental.pallas import tpu as pltpu
from jax.experimental.pallas import tpu_sc as plsc

NSUB = 16
CHUNK = 128
SK = 1
IBLK = 8
SPB = IBLK // SK
NBUF = 2
PF = 2
HID = 128
FEAT = HID // 2
ROW_BLK = 2000


def _cdiv(a, b):
    return (a + b - 1) // b


def _sinusoid_np(n, d):
    pos = np.arange(n)[:, None].astype(np.float32)
    i = np.arange(d)[None, :]
    angle = pos / np.power(10000.0, (2 * (i // 2)) / float(d))
    pe = np.where(i % 2 == 0, np.sin(angle), np.cos(angle))
    return jnp.asarray(pe, jnp.float32)


def _pad_edges(src, dst, attr, nch):
    e = src.shape[0]
    pad = NSUB * nch * CHUNK - e
    src = jnp.pad(src, (0, pad))
    dst = jnp.pad(dst, (0, pad))
    attr = jnp.pad(attr, (0, pad))
    return (src.reshape(NSUB, nch, CHUNK), dst.reshape(NSUB, nch, CHUNK),
            attr.reshape(NSUB, nch * CHUNK))


_GDN = lax.GatherDimensionNumbers(
    offset_dims=(), collapsed_slice_dims=(0,), start_index_map=(0,))


@functools.lru_cache(maxsize=None)
def _sc_layer(n_nodes, nch_w, nch_s):
    rpt = (n_nodes // NSUB) // 8 * 8
    tail = n_nodes - rpt * NSUB
    mesh = plsc.VectorSubcoreMesh(core_axis_name="c", subcore_axis_name="s")

    def body(*refs):
        (hw_l, hw_r, hs_l, hs_r, wsrc, wdst, wattr, ssrc, sdst, sattr, zeros,
         mw_l, mw_r, mf_l, mf_r, mb_l, mb_r, table, acc) = refs[:19]
        rest = list(refs[19:])
        srcs = [rest.pop(0), rest.pop(0)]
        dsts = [rest.pop(0), rest.pop(0)]
        attrs = [rest.pop(0), rest.pop(0)]
        rbs = [rest.pop(0) for _ in range(NBUF)]
        gsems = [rest.pop(0) for _ in range(NBUF)]
        ssems = [rest.pop(0) for _ in range(NBUF)]
        isems = [rest.pop(0), rest.pop(0)]
        c = lax.axis_index("c")
        s = lax.axis_index("s")
        own = pl.ds(s * rpt, rpt)
        tl = pl.ds(rpt * NSUB, tail)

        def run(h_half, src_hbm, dst_hbm, attr_hbm, out_half, nch, load_tab):
            nblk = nch // IBLK

            def stage_copies(g1, sbn):
                return [
                    pltpu.make_async_copy(
                        src_hbm.at[s, pl.ds(g1 * IBLK, IBLK)],
                        srcs[sbn], isems[sbn]),
                    pltpu.make_async_copy(
                        dst_hbm.at[s, pl.ds(g1 * IBLK, IBLK)],
                        dsts[sbn], isems[sbn]),
                    pltpu.make_async_copy(
                        attr_hbm.at[s, pl.ds(g1 * IBLK * CHUNK, IBLK * CHUNK)],
                        attrs[sbn], isems[sbn]),
                ]

            def gather_desc(idx2, b):
                return pltpu.make_async_copy(
                    table.at[idx2], rbs[b], gsems[b])

            def scatter_desc(idx2, b):
                return pltpu.make_async_copy(
                    rbs[b], acc.at[idx2], ssems[b])

            pltpu.sync_copy(zeros.at[own], acc.at[own])
            if load_tab:
                pltpu.sync_copy(h_half.at[own], table.at[own])
            if tail:
                @pl.when(s == NSUB - 1)
                def _():
                    pltpu.sync_copy(zeros.at[tl], acc.at[tl])
                    if load_tab:
                        pltpu.sync_copy(h_half.at[tl], table.at[tl])
            plsc.subcore_barrier()

            pltpu.sync_copy(src_hbm.at[s, pl.ds(0, IBLK)], srcs[0])
            pltpu.sync_copy(dst_hbm.at[s, pl.ds(0, IBLK)], dsts[0])
            pltpu.sync_copy(attr_hbm.at[s, pl.ds(0, IBLK * CHUNK)], attrs[0])
            for q in range(PF):
                gather_desc(srcs[0].at[q], q % NBUF).start()

            def process_block(g, sbi):
                sb, sbn = sbi, 1 - sbi
                have_next = g + 1 < nblk

                @pl.when(have_next)
                def _():
                    for d in stage_copies(g + 1, sbn):
                        d.start()

                for jj2 in range(SPB):
                    q = g * SPB + jj2
                    b = jj2 % NBUF
                    gather_desc(srcs[sb].at[jj2], b).wait()

                    if True:
                        abase = jj2 * CHUNK

                        @pl.loop(0, CHUNK // 16)
                        def _eg(eg):
                            av16 = attrs[sb][pl.ds(abase + eg * 16, 16)]

                            @pl.loop(0, 16, unroll=4)
                            def _l(l):
                                bc = lax.gather(
                                    av16, jnp.full((16, 1), l, jnp.int32),
                                    _GDN, (1,),
                                    mode=lax.GatherScatterMode
                                    .PROMISE_IN_BOUNDS)
                                for k in range(FEAT // 16):
                                    ix = (eg * 16 + l, pl.ds(k * 16, 16))
                                    rbs[b][ix] = rbs[b][ix] * bc

                    pltpu.async_copy(rbs[b], acc.at[dsts[sb].at[jj2]],
                                     ssems[b], add=True)

                    if jj2 == SPB - PF:
                        @pl.when(have_next)
                        def _():
                            for d in stage_copies(g + 1, sbn):
                                d.wait()

                    qn = q + PF
                    jj2n = jj2 + PF
                    bn = jj2n % NBUF
                    nidx = (srcs[sb].at[jj2n] if jj2n < SPB
                            else srcs[sbn].at[jj2n - SPB])
                    nsch = nch // SK

                    @pl.when((qn < nsch) & (q >= NBUF - PF))
                    def _():
                        scatter_desc(dsts[sb].at[jj2], bn).wait()

                    @pl.when(qn < nsch)
                    def _():
                        gather_desc(nidx, bn).start()

            @pl.loop(0, nblk // 2)
            def _g2(g2):
                process_block(g2 * 2, 0)
                process_block(g2 * 2 + 1, 1)

            for b in range(NBUF):
                scatter_desc(dsts[0].at[0], b).wait()

            plsc.subcore_barrier()
            pltpu.sync_copy(acc.at[own], out_half.at[own])
            if tail:
                @pl.when(s == NSUB - 1)
                def _():
                    pltpu.sync_copy(acc.at[tl], out_half.at[tl])

        @pl.when(c == 0)
        def _():
            for hw_h, mw_h in ((hw_l, mw_l), (hw_r, mw_r)):
                run(hw_h, wsrc, wdst, wattr, mw_h, nch_w, True)
                plsc.subcore_barrier()
                plsc.subcore_barrier()

        @pl.when(c == 1)
        def _():
            for hs_h, mf_h, mb_h in ((hs_l, mf_l, mb_l), (hs_r, mf_r, mb_r)):
                run(hs_h, ssrc, sdst, sattr, mf_h, nch_s, True)
                run(hs_h, sdst, ssrc, sattr, mb_h, nch_s, False)

    out_t = [jax.ShapeDtypeStruct((n_nodes, FEAT), jnp.float32)] * 6
    return pl.kernel(
        body,
        out_type=out_t,
        mesh=mesh,
        scratch_types=(
            [pltpu.VMEM_SHARED((n_nodes, FEAT), jnp.float32)] * 2
            + [pltpu.VMEM((IBLK, CHUNK), jnp.int32)] * 4
            + [pltpu.VMEM((IBLK * CHUNK,), jnp.float32)] * 2
            + [pltpu.VMEM((CHUNK, FEAT), jnp.float32)] * NBUF
            + [pltpu.SemaphoreType.DMA] * (2 * NBUF + 2)
        ),
    )


def _dot(a, b):
    return jnp.dot(a, b, preferred_element_type=jnp.float32)


def _silu(x):
    return x * jax.nn.sigmoid(x)


def _proj_body(wx, sx, win, sin_, pe, hwl_o, hwr_o, hsl_o, hsr_o):
    hw = _dot(wx[...], win[...])
    hs = _dot(sx[...], sin_[...]) + pe[...]
    hwl_o[...] = hw[:, :FEAT]
    hwr_o[...] = hw[:, FEAT:]
    hsl_o[...] = hs[:, :FEAT]
    hsr_o[...] = hs[:, FEAT:]


def _gru_blk(g, h, wx, wh, b):
    gx = _dot(g, wx) + b
    gh = _dot(h, wh)
    r = jax.nn.sigmoid(gx[:, :HID] + gh[:, :HID])
    z = jax.nn.sigmoid(gx[:, HID:2 * HID] + gh[:, HID:2 * HID])
    n = jnp.tanh(gx[:, 2 * HID:] + r * gh[:, 2 * HID:])
    return (1.0 - z) * n + z * h


def _dense_body(mwl, mwr, hwl, hwr, mfl, mfr, mbl, mbr, hsl, hsr,
                wg1, wg2, wwx, wwh, wb,
                sg1, sg2, fwx, fwh, fb, bwx, bwh, bb,
                hwl_o, hwr_o, hsl_o, hsr_o):
    mw = jnp.concatenate([mwl[...], mwr[...]], axis=1)
    hw = jnp.concatenate([hwl[...], hwr[...]], axis=1)
    mf = jnp.concatenate([mfl[...], mfr[...]], axis=1)
    mb = jnp.concatenate([mbl[...], mbr[...]], axis=1)
    hs = jnp.concatenate([hsl[...], hsr[...]], axis=1)
    g = _dot(mw, wg1[...]) * _silu(_dot(mw, wg2[...]))
    hw_n = _gru_blk(g, hw, wwx[...], wwh[...], wb[...])
    gf = _dot(mf, sg1[...]) * _silu(_dot(mf, sg2[...]))
    gb = _dot(mb, sg1[...]) * _silu(_dot(mb, sg2[...]))
    hf = _gru_blk(gf, hs, fwx[...], fwh[...], fb[...])
    hb = _gru_blk(gb, hs, bwx[...], bwh[...], bb[...])
    hs_n = 0.5 * (hf + hb)
    hwl_o[...] = hw_n[:, :FEAT]
    hwr_o[...] = hw_n[:, FEAT:]
    hsl_o[...] = hs_n[:, :FEAT]
    hsr_o[...] = hs_n[:, FEAT:]


def _pool_head_body(hwl, hwr, hsl, hsr, wbat, sbat, wout_w, sout_w,
                    fw1, fw2, fb, lng, lnb, c1w, c1b, c2w, c2b,
                    out, wsum, ssum, wcnt, scnt):
    i = pl.program_id(0)
    nblk = pl.num_programs(0)

    @pl.when(i == 0)
    def _():
        wsum[...] = jnp.zeros_like(wsum)
        ssum[...] = jnp.zeros_like(ssum)
        wcnt[...] = jnp.zeros_like(wcnt)
        scnt[...] = jnp.zeros_like(scnt)

    hw = jnp.concatenate([hwl[...], hwr[...]], axis=1)
    hs = jnp.concatenate([hsl[...], hsr[...]], axis=1)
    gid = lax.broadcasted_iota(jnp.int32, (64, ROW_BLK), 0)
    yw = _dot(hw, wout_w[...])
    ohw = (gid == wbat[0, 0, :][None, :]).astype(jnp.float32)
    wsum[...] += _dot(ohw, yw)
    wcnt[...] += jnp.broadcast_to(jnp.sum(ohw, axis=1, keepdims=True),
                                  wcnt.shape)
    ys = _dot(hs, sout_w[...])
    ohs = (gid == sbat[0, 0, :][None, :]).astype(jnp.float32)
    ssum[...] += _dot(ohs, ys)
    scnt[...] += jnp.broadcast_to(jnp.sum(ohs, axis=1, keepdims=True),
                                  scnt.shape)

    @pl.when(i == nblk - 1)
    def _():
        w = wsum[...] / jnp.maximum(wcnt[...], 1.0)
        so = ssum[...] / jnp.maximum(scnt[...], 1.0)
        alpha = jax.nn.sigmoid(_dot(w, fw1[...]) + _dot(so, fw2[...]) + fb[...])
        fused = alpha * w + (1.0 - alpha) * so
        mu = jnp.mean(fused, axis=-1, keepdims=True)
        xc = fused - mu
        var = jnp.mean(xc * xc, axis=-1, keepdims=True)
        xn = xc * jax.lax.rsqrt(var + 1e-5) * lng[...] + lnb[...]
        xr = jnp.maximum(_dot(xn, c1w[...]) + c1b[...], 0.0)
        out[...] = _dot(xr, c2w[...]) + c2b[...]


def kernel(word_x, word_edge_index, word_edge_attr, word_batch,
           sentence_x, sentence_edge_index, sentence_edge_attr, sentence_batch,
           params):
    p = params
    nw = word_x.shape[0]
    ns = sentence_x.shape[0]
    assert nw == ns and nw % NSUB == 0
    ew = word_edge_index.shape[1]
    es = sentence_edge_index.shape[1]
    ncls = p['c2_w'].shape[1]

    nch_w = _cdiv(ew, NSUB * CHUNK * 2 * IBLK) * 2 * IBLK
    nch_s = _cdiv(es, NSUB * CHUNK * 2 * IBLK) * 2 * IBLK
    wsrc, wdst, wattr = _pad_edges(word_edge_index[0], word_edge_index[1],
                                   word_edge_attr, nch_w)
    ssrc, sdst, sattr = _pad_edges(sentence_edge_index[0],
                                   sentence_edge_index[1],
                                   sentence_edge_attr, nch_s)
    zeros = jnp.zeros((nw, FEAT), jnp.float32)
    pe = _sinusoid_np(ns, HID)

    nblk = nw // ROW_BLK
    half_rows = lambda: pl.BlockSpec((ROW_BLK, FEAT), lambda i: (i, 0))
    grid_rows = lambda: pl.BlockSpec((ROW_BLK, HID), lambda i: (i, 0))
    full = lambda shp: pl.BlockSpec(shp, lambda i: tuple(0 for _ in shp))

    hwl, hwr, hsl, hsr = pl.pallas_call(
        _proj_body,
        grid=(nblk,),
        in_specs=[grid_rows(), grid_rows(), full((HID, HID)), full((HID, HID)),
                  grid_rows()],
        out_specs=[half_rows()] * 4,
        out_shape=[jax.ShapeDtypeStruct((nw, FEAT), jnp.float32)] * 4,
    )(word_x, sentence_x, p['w_in'], p['s_in'], pe)

    sc = _sc_layer(nw, nch_w, nch_s)
    dense = pl.pallas_call(
        _dense_body,
        grid=(nblk,),
        in_specs=[half_rows()] * 10 + [
            full((HID, HID)), full((HID, HID)),
            full((HID, 3 * HID)), full((HID, 3 * HID)), full((1, 3 * HID)),
            full((HID, HID)), full((HID, HID)),
            full((HID, 3 * HID)), full((HID, 3 * HID)), full((1, 3 * HID)),
            full((HID, 3 * HID)), full((HID, 3 * HID)), full((1, 3 * HID)),
        ],
        out_specs=[half_rows()] * 4,
        out_shape=[jax.ShapeDtypeStruct((nw, FEAT), jnp.float32)] * 4,
    )

    wb = p['w_gru_b'].reshape(1, 3 * HID)
    fbias = p['s_gru_b_f'].reshape(1, 3 * HID)
    bbias = p['s_gru_b_b'].reshape(1, 3 * HID)
    for _ in range(3):
        mwl, mwr, mfl, mfr, mbl, mbr = sc(
            hwl, hwr, hsl, hsr, wsrc, wdst, wattr, ssrc, sdst, sattr, zeros)
        hwl, hwr, hsl, hsr = dense(
            mwl, mwr, hwl, hwr, mfl, mfr, mbl, mbr, hsl, hsr,
            p['w_g1'], p['w_g2'], p['w_gru_wx'], p['w_gru_wh'], wb,
            p['s_g1'], p['s_g2'],
            p['s_gru_wx_f'], p['s_gru_wh_f'], fbias,
            p['s_gru_wx_b'], p['s_gru_wh_b'], bbias)

    c2w = jnp.zeros((HID, HID), jnp.float32).at[:, :ncls].set(p['c2_w'])
    c2b = jnp.zeros((1, HID), jnp.float32).at[0, :ncls].set(p['c2_b'])
    wbat = word_batch.reshape(nblk, 1, ROW_BLK)
    sbat = sentence_batch.reshape(nblk, 1, ROW_BLK)
    bat_spec = pl.BlockSpec((1, 1, ROW_BLK), lambda i: (i, 0, 0))

    out = pl.pallas_call(
        _pool_head_body,
        grid=(nblk,),
        in_specs=[half_rows()] * 4 + [bat_spec, bat_spec,
                  full((HID, HID)), full((HID, HID)),
                  full((HID, HID)), full((HID, HID)), full((1, HID)),
                  full((1, HID)), full((1, HID)),
                  full((HID, HID)), full((1, HID)),
                  full((HID, HID)), full((1, HID))],
        out_specs=pl.BlockSpec((64, HID), lambda i: (0, 0)),
        out_shape=jax.ShapeDtypeStruct((64, HID), jnp.float32),
        scratch_shapes=[pltpu.VMEM((64, HID), jnp.float32)] * 4,
    )(hwl, hwr, hsl, hsr, wbat, sbat, p['w_out'], p['s_out'],
      p['fuse_w'][:HID], p['fuse_w'][HID:],
      p['fuse_b'].reshape(1, HID),
      p['ln_g'].reshape(1, HID), p['ln_b'].reshape(1, HID),
      p['c1_w'], p['c1_b'].reshape(1, HID), c2w, c2b)

    return out[:, :ncls]


# speedup vs baseline: 5.1716x; 1.1039x over previous
import functools

import numpy as np
import jax
import jax.numpy as jnp
from jax import lax
from jax.experimental import pallas as pl
from jax.experimental.pallas import tpu as pltpu
from jax.experimental.pallas import tpu_sc as plsc

NSUB = 16
CHUNK = 128
SK = 1
IBLK = 8
SPB = IBLK // SK
NBUF = 2
PF = 2
HID = 128
FEAT = HID // 2
ROW_BLK = 2000


def _cdiv(a, b):
    return (a + b - 1) // b


def _sinusoid_np(n, d):
    pos = np.arange(n)[:, None].astype(np.float32)
    i = np.arange(d)[None, :]
    angle = pos / np.power(10000.0, (2 * (i // 2)) / float(d))
    pe = np.where(i % 2 == 0, np.sin(angle), np.cos(angle))
    return jnp.asarray(pe, jnp.float32)


def _pad_edges(src, dst, attr, nch, n_nodes):
    e = src.shape[0]
    pad = NSUB * nch * CHUNK - e
    spread = (np.arange(pad, dtype=np.int32) * 61) % n_nodes
    src = jnp.concatenate([src, jnp.asarray(spread)])
    dst = jnp.concatenate([dst, jnp.asarray(spread)])
    attr = jnp.pad(attr, (0, pad))
    return (src.reshape(NSUB, nch, CHUNK), dst.reshape(NSUB, nch, CHUNK),
            attr.reshape(NSUB, nch * CHUNK))


_GDN = lax.GatherDimensionNumbers(
    offset_dims=(), collapsed_slice_dims=(0,), start_index_map=(0,))


@functools.lru_cache(maxsize=None)
def _sc_layer(n_nodes, nch_w, nch_s):
    rpt = (n_nodes // NSUB) // 8 * 8
    tail = n_nodes - rpt * NSUB
    mesh = plsc.VectorSubcoreMesh(core_axis_name="c", subcore_axis_name="s")

    def body(*refs):
        (hw_l, hw_r, hs_l, hs_r, wsrc, wdst, wattr, ssrc, sdst, sattr, zeros,
         mw_l, mw_r, mf_l, mf_r, mb_l, mb_r, table, acc) = refs[:19]
        rest = list(refs[19:])
        srcs = [rest.pop(0), rest.pop(0)]
        dsts = [rest.pop(0), rest.pop(0)]
        attrs = [rest.pop(0), rest.pop(0)]
        rbs = [rest.pop(0) for _ in range(NBUF)]
        gsems = [rest.pop(0) for _ in range(NBUF)]
        ssems = [rest.pop(0) for _ in range(NBUF)]
        isems = [rest.pop(0), rest.pop(0)]
        c = lax.axis_index("c")
        s = lax.axis_index("s")
        own = pl.ds(s * rpt, rpt)
        tl = pl.ds(rpt * NSUB, tail)

        def run(h_half, src_hbm, dst_hbm, attr_hbm, out_half, nch, load_tab):
            nblk = nch // IBLK

            def stage_copies(g1, sbn):
                return [
                    pltpu.make_async_copy(
                        src_hbm.at[s, pl.ds(g1 * IBLK, IBLK)],
                        srcs[sbn], isems[sbn]),
                    pltpu.make_async_copy(
                        dst_hbm.at[s, pl.ds(g1 * IBLK, IBLK)],
                        dsts[sbn], isems[sbn]),
                    pltpu.make_async_copy(
                        attr_hbm.at[s, pl.ds(g1 * IBLK * CHUNK, IBLK * CHUNK)],
                        attrs[sbn], isems[sbn]),
                ]

            def gather_desc(idx2, b):
                return pltpu.make_async_copy(
                    table.at[idx2], rbs[b], gsems[b])

            def scatter_desc(idx2, b):
                return pltpu.make_async_copy(
                    rbs[b], acc.at[idx2], ssems[b])

            pltpu.sync_copy(zeros.at[own], acc.at[own])
            if load_tab:
                pltpu.sync_copy(h_half.at[own], table.at[own])
            if tail:
                @pl.when(s == NSUB - 1)
                def _():
                    pltpu.sync_copy(zeros.at[tl], acc.at[tl])
                    if load_tab:
                        pltpu.sync_copy(h_half.at[tl], table.at[tl])
            plsc.subcore_barrier()

            pltpu.sync_copy(src_hbm.at[s, pl.ds(0, IBLK)], srcs[0])
            pltpu.sync_copy(dst_hbm.at[s, pl.ds(0, IBLK)], dsts[0])
            pltpu.sync_copy(attr_hbm.at[s, pl.ds(0, IBLK * CHUNK)], attrs[0])
            for q in range(PF):
                gather_desc(srcs[0].at[q], q % NBUF).start()

            def process_block(g, sbi):
                sb, sbn = sbi, 1 - sbi
                have_next = g + 1 < nblk

                @pl.when(have_next)
                def _():
                    for d in stage_copies(g + 1, sbn):
                        d.start()

                for jj2 in range(SPB):
                    q = g * SPB + jj2
                    b = jj2 % NBUF
                    gather_desc(srcs[sb].at[jj2], b).wait()

                    if True:
                        abase = jj2 * CHUNK

                        @pl.loop(0, CHUNK // 16)
                        def _eg(eg):
                            av16 = attrs[sb][pl.ds(abase + eg * 16, 16)]

                            @pl.loop(0, 16, unroll=4)
                            def _l(l):
                                bc = lax.gather(
                                    av16, jnp.full((16, 1), l, jnp.int32),
                                    _GDN, (1,),
                                    mode=lax.GatherScatterMode
                                    .PROMISE_IN_BOUNDS)
                                for k in range(FEAT // 16):
                                    ix = (eg * 16 + l, pl.ds(k * 16, 16))
                                    rbs[b][ix] = rbs[b][ix] * bc

                    pltpu.async_copy(rbs[b], acc.at[dsts[sb].at[jj2]],
                                     ssems[b], add=True)

                    if jj2 == SPB - PF:
                        @pl.when(have_next)
                        def _():
                            for d in stage_copies(g + 1, sbn):
                                d.wait()

                    qn = q + PF
                    jj2n = jj2 + PF
                    bn = jj2n % NBUF
                    nidx = (srcs[sb].at[jj2n] if jj2n < SPB
                            else srcs[sbn].at[jj2n - SPB])
                    nsch = nch // SK

                    @pl.when((qn < nsch) & (q >= NBUF - PF))
                    def _():
                        scatter_desc(dsts[sb].at[jj2], bn).wait()

                    @pl.when(qn < nsch)
                    def _():
                        gather_desc(nidx, bn).start()

            @pl.loop(0, nblk // 2)
            def _g2(g2):
                process_block(g2 * 2, 0)
                process_block(g2 * 2 + 1, 1)

            for b in range(NBUF):
                scatter_desc(dsts[0].at[0], b).wait()

            plsc.subcore_barrier()
            pltpu.sync_copy(acc.at[own], out_half.at[own])
            if tail:
                @pl.when(s == NSUB - 1)
                def _():
                    pltpu.sync_copy(acc.at[tl], out_half.at[tl])

        @pl.when(c == 0)
        def _():
            for hw_h, mw_h in ((hw_l, mw_l), (hw_r, mw_r)):
                run(hw_h, wsrc, wdst, wattr, mw_h, nch_w, True)
                plsc.subcore_barrier()
                plsc.subcore_barrier()

        @pl.when(c == 1)
        def _():
            for hs_h, mf_h, mb_h in ((hs_l, mf_l, mb_l), (hs_r, mf_r, mb_r)):
                run(hs_h, ssrc, sdst, sattr, mf_h, nch_s, True)
                run(hs_h, sdst, ssrc, sattr, mb_h, nch_s, False)

    out_t = [jax.ShapeDtypeStruct((n_nodes, FEAT), jnp.float32)] * 6
    return pl.kernel(
        body,
        out_type=out_t,
        mesh=mesh,
        scratch_types=(
            [pltpu.VMEM_SHARED((n_nodes, FEAT), jnp.float32)] * 2
            + [pltpu.VMEM((IBLK, CHUNK), jnp.int32)] * 4
            + [pltpu.VMEM((IBLK * CHUNK,), jnp.float32)] * 2
            + [pltpu.VMEM((CHUNK, FEAT), jnp.float32)] * NBUF
            + [pltpu.SemaphoreType.DMA] * (2 * NBUF + 2)
        ),
    )


def _dot(a, b):
    return jnp.dot(a, b, preferred_element_type=jnp.float32)


def _silu(x):
    return x * jax.nn.sigmoid(x)


def _proj_body(wx, sx, win, sin_, pe, hwl_o, hwr_o, hsl_o, hsr_o):
    hw = _dot(wx[...], win[...])
    hs = _dot(sx[...], sin_[...]) + pe[...]
    hwl_o[...] = hw[:, :FEAT]
    hwr_o[...] = hw[:, FEAT:]
    hsl_o[...] = hs[:, :FEAT]
    hsr_o[...] = hs[:, FEAT:]


def _gru_blk(g, h, wx, wh, b):
    gx = _dot(g, wx) + b
    gh = _dot(h, wh)
    r = jax.nn.sigmoid(gx[:, :HID] + gh[:, :HID])
    z = jax.nn.sigmoid(gx[:, HID:2 * HID] + gh[:, HID:2 * HID])
    n = jnp.tanh(gx[:, 2 * HID:] + r * gh[:, 2 * HID:])
    return (1.0 - z) * n + z * h


def _dense_body(mwl, mwr, hwl, hwr, mfl, mfr, mbl, mbr, hsl, hsr,
                wg1, wg2, wwx, wwh, wb,
                sg1, sg2, fwx, fwh, fb, bwx, bwh, bb,
                hwl_o, hwr_o, hsl_o, hsr_o):
    mw = jnp.concatenate([mwl[...], mwr[...]], axis=1)
    hw = jnp.concatenate([hwl[...], hwr[...]], axis=1)
    mf = jnp.concatenate([mfl[...], mfr[...]], axis=1)
    mb = jnp.concatenate([mbl[...], mbr[...]], axis=1)
    hs = jnp.concatenate([hsl[...], hsr[...]], axis=1)
    g = _dot(mw, wg1[...]) * _silu(_dot(mw, wg2[...]))
    hw_n = _gru_blk(g, hw, wwx[...], wwh[...], wb[...])
    gf = _dot(mf, sg1[...]) * _silu(_dot(mf, sg2[...]))
    gb = _dot(mb, sg1[...]) * _silu(_dot(mb, sg2[...]))
    hf = _gru_blk(gf, hs, fwx[...], fwh[...], fb[...])
    hb = _gru_blk(gb, hs, bwx[...], bwh[...], bb[...])
    hs_n = 0.5 * (hf + hb)
    hwl_o[...] = hw_n[:, :FEAT]
    hwr_o[...] = hw_n[:, FEAT:]
    hsl_o[...] = hs_n[:, :FEAT]
    hsr_o[...] = hs_n[:, FEAT:]


def _pool_head_body(hwl, hwr, hsl, hsr, wbat, sbat, wout_w, sout_w,
                    fw1, fw2, fb, lng, lnb, c1w, c1b, c2w, c2b,
                    out, wsum, ssum, wcnt, scnt):
    i = pl.program_id(0)
    nblk = pl.num_programs(0)

    @pl.when(i == 0)
    def _():
        wsum[...] = jnp.zeros_like(wsum)
        ssum[...] = jnp.zeros_like(ssum)
        wcnt[...] = jnp.zeros_like(wcnt)
        scnt[...] = jnp.zeros_like(scnt)

    hw = jnp.concatenate([hwl[...], hwr[...]], axis=1)
    hs = jnp.concatenate([hsl[...], hsr[...]], axis=1)
    gid = lax.broadcasted_iota(jnp.int32, (64, ROW_BLK), 0)
    yw = _dot(hw, wout_w[...])
    ohw = (gid == wbat[0, 0, :][None, :]).astype(jnp.float32)
    wsum[...] += _dot(ohw, yw)
    wcnt[...] += jnp.broadcast_to(jnp.sum(ohw, axis=1, keepdims=True),
                                  wcnt.shape)
    ys = _dot(hs, sout_w[...])
    ohs = (gid == sbat[0, 0, :][None, :]).astype(jnp.float32)
    ssum[...] += _dot(ohs, ys)
    scnt[...] += jnp.broadcast_to(jnp.sum(ohs, axis=1, keepdims=True),
                                  scnt.shape)

    @pl.when(i == nblk - 1)
    def _():
        w = wsum[...] / jnp.maximum(wcnt[...], 1.0)
        so = ssum[...] / jnp.maximum(scnt[...], 1.0)
        alpha = jax.nn.sigmoid(_dot(w, fw1[...]) + _dot(so, fw2[...]) + fb[...])
        fused = alpha * w + (1.0 - alpha) * so
        mu = jnp.mean(fused, axis=-1, keepdims=True)
        xc = fused - mu
        var = jnp.mean(xc * xc, axis=-1, keepdims=True)
        xn = xc * jax.lax.rsqrt(var + 1e-5) * lng[...] + lnb[...]
        xr = jnp.maximum(_dot(xn, c1w[...]) + c1b[...], 0.0)
        out[...] = _dot(xr, c2w[...]) + c2b[...]


def kernel(word_x, word_edge_index, word_edge_attr, word_batch,
           sentence_x, sentence_edge_index, sentence_edge_attr, sentence_batch,
           params):
    p = params
    nw = word_x.shape[0]
    ns = sentence_x.shape[0]
    assert nw == ns and nw % NSUB == 0
    ew = word_edge_index.shape[1]
    es = sentence_edge_index.shape[1]
    ncls = p['c2_w'].shape[1]

    nch_w = _cdiv(ew, NSUB * CHUNK * 2 * IBLK) * 2 * IBLK
    nch_s = _cdiv(es, NSUB * CHUNK * 2 * IBLK) * 2 * IBLK
    wsrc, wdst, wattr = _pad_edges(word_edge_index[0], word_edge_index[1],
                                   word_edge_attr, nch_w, nw)
    ssrc, sdst, sattr = _pad_edges(sentence_edge_index[0],
                                   sentence_edge_index[1],
                                   sentence_edge_attr, nch_s, ns)
    zeros = jnp.zeros((nw, FEAT), jnp.float32)
    pe = _sinusoid_np(ns, HID)

    nblk = nw // ROW_BLK
    half_rows = lambda: pl.BlockSpec((ROW_BLK, FEAT), lambda i: (i, 0))
    grid_rows = lambda: pl.BlockSpec((ROW_BLK, HID), lambda i: (i, 0))
    full = lambda shp: pl.BlockSpec(shp, lambda i: tuple(0 for _ in shp))

    hwl, hwr, hsl, hsr = pl.pallas_call(
        _proj_body,
        grid=(nblk,),
        in_specs=[grid_rows(), grid_rows(), full((HID, HID)), full((HID, HID)),
                  grid_rows()],
        out_specs=[half_rows()] * 4,
        out_shape=[jax.ShapeDtypeStruct((nw, FEAT), jnp.float32)] * 4,
    )(word_x, sentence_x, p['w_in'], p['s_in'], pe)

    sc = _sc_layer(nw, nch_w, nch_s)
    dense = pl.pallas_call(
        _dense_body,
        grid=(nblk,),
        in_specs=[half_rows()] * 10 + [
            full((HID, HID)), full((HID, HID)),
            full((HID, 3 * HID)), full((HID, 3 * HID)), full((1, 3 * HID)),
            full((HID, HID)), full((HID, HID)),
            full((HID, 3 * HID)), full((HID, 3 * HID)), full((1, 3 * HID)),
            full((HID, 3 * HID)), full((HID, 3 * HID)), full((1, 3 * HID)),
        ],
        out_specs=[half_rows()] * 4,
        out_shape=[jax.ShapeDtypeStruct((nw, FEAT), jnp.float32)] * 4,
    )

    wb = p['w_gru_b'].reshape(1, 3 * HID)
    fbias = p['s_gru_b_f'].reshape(1, 3 * HID)
    bbias = p['s_gru_b_b'].reshape(1, 3 * HID)
    for _ in range(3):
        mwl, mwr, mfl, mfr, mbl, mbr = sc(
            hwl, hwr, hsl, hsr, wsrc, wdst, wattr, ssrc, sdst, sattr, zeros)
        hwl, hwr, hsl, hsr = dense(
            mwl, mwr, hwl, hwr, mfl, mfr, mbl, mbr, hsl, hsr,
            p['w_g1'], p['w_g2'], p['w_gru_wx'], p['w_gru_wh'], wb,
            p['s_g1'], p['s_g2'],
            p['s_gru_wx_f'], p['s_gru_wh_f'], fbias,
            p['s_gru_wx_b'], p['s_gru_wh_b'], bbias)

    c2w = jnp.zeros((HID, HID), jnp.float32).at[:, :ncls].set(p['c2_w'])
    c2b = jnp.zeros((1, HID), jnp.float32).at[0, :ncls].set(p['c2_b'])
    wbat = word_batch.reshape(nblk, 1, ROW_BLK)
    sbat = sentence_batch.reshape(nblk, 1, ROW_BLK)
    bat_spec = pl.BlockSpec((1, 1, ROW_BLK), lambda i: (i, 0, 0))

    out = pl.pallas_call(
        _pool_head_body,
        grid=(nblk,),
        in_specs=[half_rows()] * 4 + [bat_spec, bat_spec,
                  full((HID, HID)), full((HID, HID)),
                  full((HID, HID)), full((HID, HID)), full((1, HID)),
                  full((1, HID)), full((1, HID)),
                  full((HID, HID)), full((1, HID)),
                  full((HID, HID)), full((1, HID))],
        out_specs=pl.BlockSpec((64, HID), lambda i: (0, 0)),
        out_shape=jax.ShapeDtypeStruct((64, HID), jnp.float32),
        scratch_shapes=[pltpu.VMEM((64, HID), jnp.float32)] * 4,
    )(hwl, hwr, hsl, hsr, wbat, sbat, p['w_out'], p['s_out'],
      p['fuse_w'][:HID], p['fuse_w'][HID:],
      p['fuse_b'].reshape(1, HID),
      p['ln_g'].reshape(1, HID), p['ln_b'].reshape(1, HID),
      p['c1_w'], p['c1_b'].reshape(1, HID), c2w, c2b)

    return out[:, :ncls]


# speedup vs baseline: 6.9322x; 1.3404x over previous
import functools

import numpy as np
import jax
import jax.numpy as jnp
from jax import lax
from jax.experimental import pallas as pl
from jax.experimental.pallas import tpu as pltpu
from jax.experimental.pallas import tpu_sc as plsc

NSUB = 16
CHUNK = 128
IBLK = 4
NBUF = 2
PF = 2
HID = 128
ROW_BLK = 2000


def _cdiv(a, b):
    return (a + b - 1) // b


def _sinusoid_np(n, d):
    pos = np.arange(n)[:, None].astype(np.float32)
    i = np.arange(d)[None, :]
    angle = pos / np.power(10000.0, (2 * (i // 2)) / float(d))
    pe = np.where(i % 2 == 0, np.sin(angle), np.cos(angle))
    return jnp.asarray(pe, jnp.float32)


def _pad_edges(src, dst, attr, nch, n_nodes=10000):
    e = src.shape[0]
    pad = NSUB * nch * CHUNK - e
    spread = jnp.asarray((np.arange(pad, dtype=np.int32) * 61) % n_nodes)
    src = jnp.concatenate([src, spread])
    dst = jnp.concatenate([dst, spread])
    attr = jnp.pad(attr, (0, pad))
    return (src.reshape(NSUB, nch, CHUNK), dst.reshape(NSUB, nch, CHUNK),
            attr.reshape(NSUB, nch * CHUNK))


_GDN = lax.GatherDimensionNumbers(
    offset_dims=(), collapsed_slice_dims=(0,), start_index_map=(0,))


@functools.lru_cache(maxsize=None)
def _sc_layer(n_nodes, nch_w, nch_s):
    rpt = (n_nodes // NSUB) // 8 * 8
    tail = n_nodes - rpt * NSUB
    mesh = plsc.VectorSubcoreMesh(core_axis_name="c", subcore_axis_name="s")
    nch_max = max(nch_w, nch_s)

    def body(*refs):
        (hw, wsrc, wdst, wattr, hs, ssrc, sdst, sattr, zeros,
         m_w, m_f, m_b, acc) = refs[:13]
        rest = list(refs[13:])
        srcs = [rest.pop(0), rest.pop(0)]
        dsts = [rest.pop(0), rest.pop(0)]
        attrs = [rest.pop(0), rest.pop(0)]
        rbs = [rest.pop(0) for _ in range(NBUF)]
        gsems = [rest.pop(0) for _ in range(NBUF)]
        ssems = [rest.pop(0) for _ in range(NBUF)]
        isems = [rest.pop(0), rest.pop(0)]
        c = lax.axis_index("c")
        s = lax.axis_index("s")
        own = pl.ds(s * rpt, rpt)
        tl = pl.ds(rpt * NSUB, tail)

        def run(h_hbm, src_hbm, dst_hbm, attr_hbm, out_hbm, nch):
            nblk = nch // IBLK

            def stage_copies(g1, sbn):
                return [
                    pltpu.make_async_copy(
                        src_hbm.at[s, pl.ds(g1 * IBLK, IBLK)],
                        srcs[sbn], isems[sbn]),
                    pltpu.make_async_copy(
                        dst_hbm.at[s, pl.ds(g1 * IBLK, IBLK)],
                        dsts[sbn], isems[sbn]),
                    pltpu.make_async_copy(
                        attr_hbm.at[s, pl.ds(g1 * IBLK * CHUNK, IBLK * CHUNK)],
                        attrs[sbn], isems[sbn]),
                ]

            pltpu.sync_copy(zeros.at[own], acc.at[own])
            if tail:
                @pl.when(s == NSUB - 1)
                def _():
                    pltpu.sync_copy(zeros.at[tl], acc.at[tl])
            plsc.subcore_barrier()

            pltpu.sync_copy(src_hbm.at[s, pl.ds(0, IBLK)], srcs[0])
            pltpu.sync_copy(dst_hbm.at[s, pl.ds(0, IBLK)], dsts[0])
            pltpu.sync_copy(attr_hbm.at[s, pl.ds(0, IBLK * CHUNK)], attrs[0])
            for jj in range(PF):
                pltpu.async_copy(h_hbm.at[srcs[0].at[jj]], rbs[jj], gsems[jj])

            def process_block(g, sbi):
                sb, sbn = sbi, 1 - sbi
                have_next = g + 1 < nblk

                @pl.when(have_next)
                def _():
                    for d in stage_copies(g + 1, sbn):
                        d.start()

                for jj in range(IBLK):
                    j = g * IBLK + jj
                    b = jj % NBUF
                    pltpu.make_async_copy(
                        h_hbm.at[srcs[sb].at[jj]], rbs[b], gsems[b]).wait()

                    @pl.loop(0, CHUNK // 16)
                    def _eg(eg):
                        av16 = attrs[sb][pl.ds(jj * CHUNK + eg * 16, 16)]

                        @pl.loop(0, 16, unroll=4)
                        def _l(l):
                            bc = lax.gather(
                                av16, jnp.full((16, 1), l, jnp.int32),
                                _GDN, (1,),
                                mode=lax.GatherScatterMode.PROMISE_IN_BOUNDS)
                            for k in range(HID // 16):
                                ix = (eg * 16 + l, pl.ds(k * 16, 16))
                                rbs[b][ix] = rbs[b][ix] * bc

                    pltpu.async_copy(rbs[b], acc.at[dsts[sb].at[jj]],
                                     ssems[b], add=True)

                    if jj == IBLK - PF:
                        @pl.when(have_next)
                        def _():
                            for d in stage_copies(g + 1, sbn):
                                d.wait()

                    jn = j + PF
                    jjn = jj + PF
                    bn = jjn % NBUF
                    nsrc = (srcs[sb].at[jjn] if jjn < IBLK
                            else srcs[sbn].at[jjn - IBLK])

                    @pl.when((jn < nch) & (j >= NBUF - PF))
                    def _():
                        pltpu.make_async_copy(
                            rbs[bn], acc.at[dsts[sb].at[jj]],
                            ssems[bn]).wait()

                    @pl.when(jn < nch)
                    def _():
                        pltpu.async_copy(h_hbm.at[nsrc], rbs[bn], gsems[bn])

            @pl.loop(0, nblk // 2)
            def _g2(g2):
                process_block(g2 * 2, 0)
                process_block(g2 * 2 + 1, 1)

            for b in range(NBUF):
                pltpu.make_async_copy(
                    rbs[b], acc.at[dsts[0].at[0]], ssems[b]).wait()

            plsc.subcore_barrier()
            pltpu.sync_copy(acc.at[own], out_hbm.at[own])
            if tail:
                @pl.when(s == NSUB - 1)
                def _():
                    pltpu.sync_copy(acc.at[tl], out_hbm.at[tl])

        @pl.when(c == 0)
        def _():
            run(hw, wsrc, wdst, wattr, m_w, nch_w)
            plsc.subcore_barrier()
            plsc.subcore_barrier()

        @pl.when(c == 1)
        def _():
            run(hs, ssrc, sdst, sattr, m_f, nch_s)
            run(hs, sdst, ssrc, sattr, m_b, nch_s)

    out_t = [jax.ShapeDtypeStruct((n_nodes, HID), jnp.float32)] * 3
    return pl.kernel(
        body,
        out_type=out_t,
        mesh=mesh,
        scratch_types=(
            [pltpu.VMEM_SHARED((n_nodes, HID), jnp.float32)]
            + [pltpu.VMEM((IBLK, CHUNK), jnp.int32)] * 4
            + [pltpu.VMEM((IBLK * CHUNK,), jnp.float32)] * 2
            + [pltpu.VMEM((CHUNK, HID), jnp.float32)] * NBUF
            + [pltpu.SemaphoreType.DMA] * (2 * NBUF + 2)
        ),
    )


def _dot(a, b):
    return jnp.dot(a, b, preferred_element_type=jnp.float32)


def _silu(x):
    return x * jax.nn.sigmoid(x)


def _proj_body(wx, sx, win, sin_, pe, hw_o, hs_o):
    hw_o[...] = _dot(wx[...], win[...])
    hs_o[...] = _dot(sx[...], sin_[...]) + pe[...]


def _gru_blk(g, h, wx, wh, b):
    gx = _dot(g, wx) + b
    gh = _dot(h, wh)
    r = jax.nn.sigmoid(gx[:, :HID] + gh[:, :HID])
    z = jax.nn.sigmoid(gx[:, HID:2 * HID] + gh[:, HID:2 * HID])
    n = jnp.tanh(gx[:, 2 * HID:] + r * gh[:, 2 * HID:])
    return (1.0 - z) * n + z * h


def _dense_body(mw, hw, mf, mb, hs,
                wg1, wg2, wwx, wwh, wb,
                sg1, sg2, fwx, fwh, fb, bwx, bwh, bb,
                hw_o, hs_o):
    g = _dot(mw[...], wg1[...]) * _silu(_dot(mw[...], wg2[...]))
    hw_o[...] = _gru_blk(g, hw[...], wwx[...], wwh[...], wb[...])
    gf = _dot(mf[...], sg1[...]) * _silu(_dot(mf[...], sg2[...]))
    gb = _dot(mb[...], sg1[...]) * _silu(_dot(mb[...], sg2[...]))
    hf = _gru_blk(gf, hs[...], fwx[...], fwh[...], fb[...])
    hb = _gru_blk(gb, hs[...], bwx[...], bwh[...], bb[...])
    hs_o[...] = 0.5 * (hf + hb)


def _pool_head_body(hw, hs, wbat, sbat, wout_w, sout_w,
                    fw1, fw2, fb, lng, lnb, c1w, c1b, c2w, c2b,
                    out, wsum, ssum, wcnt, scnt):
    i = pl.program_id(0)
    nblk = pl.num_programs(0)

    @pl.when(i == 0)
    def _():
        wsum[...] = jnp.zeros_like(wsum)
        ssum[...] = jnp.zeros_like(ssum)
        wcnt[...] = jnp.zeros_like(wcnt)
        scnt[...] = jnp.zeros_like(scnt)

    gid = lax.broadcasted_iota(jnp.int32, (64, ROW_BLK), 0)
    yw = _dot(hw[...], wout_w[...])
    ohw = (gid == wbat[0, 0, :][None, :]).astype(jnp.float32)
    wsum[...] += _dot(ohw, yw)
    wcnt[...] += jnp.broadcast_to(jnp.sum(ohw, axis=1, keepdims=True), wcnt.shape)
    ys = _dot(hs[...], sout_w[...])
    ohs = (gid == sbat[0, 0, :][None, :]).astype(jnp.float32)
    ssum[...] += _dot(ohs, ys)
    scnt[...] += jnp.broadcast_to(jnp.sum(ohs, axis=1, keepdims=True), scnt.shape)

    @pl.when(i == nblk - 1)
    def _():
        w = wsum[...] / jnp.maximum(wcnt[...], 1.0)
        so = ssum[...] / jnp.maximum(scnt[...], 1.0)
        alpha = jax.nn.sigmoid(_dot(w, fw1[...]) + _dot(so, fw2[...]) + fb[...])
        fused = alpha * w + (1.0 - alpha) * so
        mu = jnp.mean(fused, axis=-1, keepdims=True)
        xc = fused - mu
        var = jnp.mean(xc * xc, axis=-1, keepdims=True)
        xn = xc * jax.lax.rsqrt(var + 1e-5) * lng[...] + lnb[...]
        xr = jnp.maximum(_dot(xn, c1w[...]) + c1b[...], 0.0)
        out[...] = _dot(xr, c2w[...]) + c2b[...]


def kernel(word_x, word_edge_index, word_edge_attr, word_batch,
           sentence_x, sentence_edge_index, sentence_edge_attr, sentence_batch,
           params):
    p = params
    nw = word_x.shape[0]
    ns = sentence_x.shape[0]
    assert nw == ns and nw % NSUB == 0
    ew = word_edge_index.shape[1]
    es = sentence_edge_index.shape[1]
    ncls = p['c2_w'].shape[1]

    nch_w = _cdiv(ew, NSUB * CHUNK * 2 * IBLK) * 2 * IBLK
    nch_s = _cdiv(es, NSUB * CHUNK * 2 * IBLK) * 2 * IBLK
    wsrc, wdst, wattr = _pad_edges(word_edge_index[0], word_edge_index[1],
                                   word_edge_attr, nch_w, nw)
    ssrc, sdst, sattr = _pad_edges(sentence_edge_index[0],
                                   sentence_edge_index[1],
                                   sentence_edge_attr, nch_s, ns)
    zeros = jnp.zeros((nw, HID), jnp.float32)
    pe = _sinusoid_np(ns, HID)

    nblk = nw // ROW_BLK
    grid_rows = lambda: pl.BlockSpec((ROW_BLK, HID), lambda i: (i, 0))
    full = lambda shp: pl.BlockSpec(shp, lambda i: tuple(0 for _ in shp))

    hw, hs = pl.pallas_call(
        _proj_body,
        grid=(nblk,),
        in_specs=[grid_rows(), grid_rows(), full((HID, HID)), full((HID, HID)),
                  grid_rows()],
        out_specs=[grid_rows(), grid_rows()],
        out_shape=[jax.ShapeDtypeStruct((nw, HID), jnp.float32)] * 2,
    )(word_x, sentence_x, p['w_in'], p['s_in'], pe)

    sc = _sc_layer(nw, nch_w, nch_s)
    dense = pl.pallas_call(
        _dense_body,
        grid=(nblk,),
        in_specs=[grid_rows()] * 5 + [
            full((HID, HID)), full((HID, HID)),
            full((HID, 3 * HID)), full((HID, 3 * HID)), full((1, 3 * HID)),
            full((HID, HID)), full((HID, HID)),
            full((HID, 3 * HID)), full((HID, 3 * HID)), full((1, 3 * HID)),
            full((HID, 3 * HID)), full((HID, 3 * HID)), full((1, 3 * HID)),
        ],
        out_specs=[grid_rows(), grid_rows()],
        out_shape=[jax.ShapeDtypeStruct((nw, HID), jnp.float32)] * 2,
    )

    wb = p['w_gru_b'].reshape(1, 3 * HID)
    fbias = p['s_gru_b_f'].reshape(1, 3 * HID)
    bbias = p['s_gru_b_b'].reshape(1, 3 * HID)
    for _ in range(3):
        m_w, m_f, m_b = sc(hw, wsrc, wdst, wattr, hs, ssrc, sdst, sattr, zeros)
        hw, hs = dense(m_w, hw, m_f, m_b, hs,
                       p['w_g1'], p['w_g2'], p['w_gru_wx'], p['w_gru_wh'], wb,
                       p['s_g1'], p['s_g2'],
                       p['s_gru_wx_f'], p['s_gru_wh_f'], fbias,
                       p['s_gru_wx_b'], p['s_gru_wh_b'], bbias)

    c2w = jnp.zeros((HID, HID), jnp.float32).at[:, :ncls].set(p['c2_w'])
    c2b = jnp.zeros((1, HID), jnp.float32).at[0, :ncls].set(p['c2_b'])
    wbat = word_batch.reshape(nblk, 1, ROW_BLK)
    sbat = sentence_batch.reshape(nblk, 1, ROW_BLK)
    bat_spec = pl.BlockSpec((1, 1, ROW_BLK), lambda i: (i, 0, 0))

    out = pl.pallas_call(
        _pool_head_body,
        grid=(nblk,),
        in_specs=[grid_rows(), grid_rows(), bat_spec, bat_spec,
                  full((HID, HID)), full((HID, HID)),
                  full((HID, HID)), full((HID, HID)), full((1, HID)),
                  full((1, HID)), full((1, HID)),
                  full((HID, HID)), full((1, HID)),
                  full((HID, HID)), full((1, HID))],
        out_specs=pl.BlockSpec((64, HID), lambda i: (0, 0)),
        out_shape=jax.ShapeDtypeStruct((64, HID), jnp.float32),
        scratch_shapes=[pltpu.VMEM((64, HID), jnp.float32)] * 4,
    )(hw, hs, wbat, sbat, p['w_out'], p['s_out'],
      p['fuse_w'][:HID], p['fuse_w'][HID:],
      p['fuse_b'].reshape(1, HID),
      p['ln_g'].reshape(1, HID), p['ln_b'].reshape(1, HID),
      p['c1_w'], p['c1_b'].reshape(1, HID), c2w, c2b)

    return out[:, :ncls]
